# Initial kernel scaffold; baseline (speedup 1.0000x reference)
import jax
import jax.numpy as jnp
from jax.experimental import pallas as pl


def kernel(feature_map, top_k):
    raise NotImplementedError("write your pallas kernel here")


# speedup vs baseline: 3.4416x; 3.4416x over previous
import jax
import jax.numpy as jnp
from jax import lax
from jax.experimental import pallas as pl
from jax.experimental.pallas import tpu as pltpu

B, C, H, W = 64, 512, 28, 28
HW = H * W
K = 9
NEG = float("-inf")


def _reduce_kernel(x_ref, max_ref, idx_ref):
    x = x_ref[...]
    m = jnp.max(x, axis=-1)
    iota = lax.broadcasted_iota(jnp.int32, x.shape, 2)
    idx = jnp.min(jnp.where(x == m[..., None], iota, HW), axis=-1)
    max_ref[...] = m
    idx_ref[...] = idx


def _select_kernel(max_ref, idx_ref, chan_ref, pos_ref, val_ref):
    maxv = max_ref[...]
    argp = idx_ref[...]
    ci = jnp.sum(maxv, axis=0, keepdims=True) * jnp.float32(1.0 / B)
    iota_c = lax.broadcasted_iota(jnp.int32, (1, C), 1)
    iota_c2 = lax.broadcasted_iota(jnp.int32, (B, C), 1)
    iota_b = lax.broadcasted_iota(jnp.int32, (B, 1), 0)
    for k in range(K):
        m = jnp.max(ci)
        c_k = jnp.min(jnp.where(ci == m, iota_c, C))
        ci = jnp.where(iota_c == c_k, NEG, ci)
        chan_ref[0, k] = c_k
        colmask = iota_c2 == c_k
        act = jnp.max(jnp.where(colmask, maxv, NEG), axis=1, keepdims=True)
        posc = jnp.max(jnp.where(colmask, argp, 0), axis=1, keepdims=True)
        for r in range(K):
            m2 = jnp.max(act)
            b_r = jnp.min(jnp.where(act == m2, iota_b, B))
            val_ref[k, r] = m2
            pos_ref[k, r] = jnp.max(jnp.where(iota_b == b_r, posc, 0))
            act = jnp.where(iota_b == b_r, NEG, act)


def _write_kernel(chan_ref, pos_ref, val_ref, out_ref):
    i = pl.program_id(0)
    k = i // K
    r = i % K
    c = chan_ref[0, k]
    p = pos_ref[k, r]
    v = val_ref[k, r]
    ci2 = lax.broadcasted_iota(jnp.int32, (C, HW), 0)
    pi2 = lax.broadcasted_iota(jnp.int32, (C, HW), 1)
    out_ref[0] = jnp.where((ci2 == c) & (pi2 == p), v, jnp.float32(0.0))


def kernel(feature_map, top_k):
    x = feature_map.reshape(B, C, HW)

    maxv, argp = pl.pallas_call(
        _reduce_kernel,
        grid=(B // 8, C // 128),
        in_specs=[pl.BlockSpec((8, 128, HW), lambda i, j: (i, j, 0))],
        out_specs=[
            pl.BlockSpec((8, 128), lambda i, j: (i, j)),
            pl.BlockSpec((8, 128), lambda i, j: (i, j)),
        ],
        out_shape=[
            jax.ShapeDtypeStruct((B, C), jnp.float32),
            jax.ShapeDtypeStruct((B, C), jnp.int32),
        ],
    )(x)

    chan, pos, val = pl.pallas_call(
        _select_kernel,
        in_specs=[
            pl.BlockSpec((B, C), lambda: (0, 0)),
            pl.BlockSpec((B, C), lambda: (0, 0)),
        ],
        out_specs=[
            pl.BlockSpec(memory_space=pltpu.SMEM),
            pl.BlockSpec(memory_space=pltpu.SMEM),
            pl.BlockSpec(memory_space=pltpu.SMEM),
        ],
        out_shape=[
            jax.ShapeDtypeStruct((1, K), jnp.int32),
            jax.ShapeDtypeStruct((K, K), jnp.int32),
            jax.ShapeDtypeStruct((K, K), jnp.float32),
        ],
    )(maxv, argp)

    out = pl.pallas_call(
        _write_kernel,
        grid=(K * K,),
        in_specs=[
            pl.BlockSpec(memory_space=pltpu.SMEM),
            pl.BlockSpec(memory_space=pltpu.SMEM),
            pl.BlockSpec(memory_space=pltpu.SMEM),
        ],
        out_specs=pl.BlockSpec((1, C, HW), lambda i: (i, 0, 0)),
        out_shape=jax.ShapeDtypeStruct((K * K, C, HW), jnp.float32),
    )(chan, pos, val)

    return out.reshape(K, K, C, H, W)
